# Initial kernel scaffold; baseline (speedup 1.0000x reference)
import jax
import jax.numpy as jnp
from jax.experimental import pallas as pl


def kernel(dists, edge_index, rels, mask, edge_mask, r_query_embed, conf_embeds, params):
    raise NotImplementedError("write your pallas kernel here")


# speedup vs baseline: 7.6068x; 7.6068x over previous
import functools

import jax
import jax.numpy as jnp
from jax.experimental import pallas as pl
from jax.experimental.pallas import tpu as pltpu
from jax.experimental.pallas import tpu_sc as plsc

B, N, E, D, NREL, M, NL = 32, 2048, 16384, 128, 500, 20, 3
_INTERP = False


def _relprep_body(rel_ref, w3_ref, out_ref):
    r = rel_ref[...]
    r2 = jnp.dot(r, w3_ref[0], preferred_element_type=jnp.float32)
    out_ref[0] = jnp.concatenate([r, r2], axis=1)


def _relprep(rel_pad, w3_all):
    nrelp = rel_pad.shape[0]
    return pl.pallas_call(
        _relprep_body,
        grid=(NL,),
        in_specs=[
            pl.BlockSpec((nrelp, D), lambda k: (0, 0)),
            pl.BlockSpec((1, D, D), lambda k: (k, 0, 0)),
        ],
        out_specs=pl.BlockSpec((1, nrelp, 2 * D), lambda k: (k, 0, 0)),
        out_shape=jax.ShapeDtypeStruct((NL, nrelp, 2 * D), jnp.float32),
        interpret=_INTERP,
    )(rel_pad, w3_all)


def _dist_emb(dists_row, de_pad_ref):
    de = jnp.zeros((N, D), jnp.float32)
    dd2 = dists_row[:, None]
    for r in range(10):
        cond = jnp.broadcast_to(dd2 == r, (N, D))
        row = jnp.broadcast_to(de_pad_ref[r:r + 1, :], (N, D))
        de = jnp.where(cond, row, de)
    return de


def _pre0_body(dists_ref, noise_ref, de_ref, w1_ref, w2_ref, b_ref, out_ref):
    de = _dist_emb(dists_ref[0, 0], de_ref)
    h = de + noise_ref[0]
    p = (jnp.dot(h, w1_ref[...], preferred_element_type=jnp.float32)
         + jnp.dot(de, w2_ref[...], preferred_element_type=jnp.float32)
         + b_ref[...])
    out_ref[0] = jnp.concatenate([h, p], axis=1)


def _pre0(dists, noise, de_pad, w1, w2, b):
    return pl.pallas_call(
        _pre0_body,
        grid=(B,),
        in_specs=[
            pl.BlockSpec((1, 1, N), lambda i: (i, 0, 0)),
            pl.BlockSpec((1, N, D), lambda i: (i, 0, 0)),
            pl.BlockSpec((16, D), lambda i: (0, 0)),
            pl.BlockSpec((D, D), lambda i: (0, 0)),
            pl.BlockSpec((D, D), lambda i: (0, 0)),
            pl.BlockSpec((1, D), lambda i: (0, 0)),
        ],
        out_specs=pl.BlockSpec((1, N, 2 * D), lambda i: (i, 0, 0)),
        out_shape=jax.ShapeDtypeStruct((B, N, 2 * D), jnp.float32),
        interpret=_INTERP,
    )(dists, noise, de_pad, w1, w2, b)


def _updpre_body(aggr_ref, tprev_ref, dists_ref, de_ref, uw_ref, ub_ref,
                 w1_ref, w2_ref, b_ref, out_ref):
    h_prev = tprev_ref[0, :, :D]
    h = (jnp.dot(aggr_ref[0], uw_ref[...], preferred_element_type=jnp.float32)
         + ub_ref[...] + h_prev)
    de = _dist_emb(dists_ref[0, 0], de_ref)
    p = (jnp.dot(h, w1_ref[...], preferred_element_type=jnp.float32)
         + jnp.dot(de, w2_ref[...], preferred_element_type=jnp.float32)
         + b_ref[...])
    out_ref[0] = jnp.concatenate([h, p], axis=1)


def _updpre(aggr, tprev, dists, de_pad, uw, ub, w1, w2, b):
    return pl.pallas_call(
        _updpre_body,
        grid=(B,),
        in_specs=[
            pl.BlockSpec((1, N, D), lambda i: (i, 0, 0)),
            pl.BlockSpec((1, N, 2 * D), lambda i: (i, 0, 0)),
            pl.BlockSpec((1, 1, N), lambda i: (i, 0, 0)),
            pl.BlockSpec((16, D), lambda i: (0, 0)),
            pl.BlockSpec((D, D), lambda i: (0, 0)),
            pl.BlockSpec((1, D), lambda i: (0, 0)),
            pl.BlockSpec((D, D), lambda i: (0, 0)),
            pl.BlockSpec((D, D), lambda i: (0, 0)),
            pl.BlockSpec((1, D), lambda i: (0, 0)),
        ],
        out_specs=pl.BlockSpec((1, N, 2 * D), lambda i: (i, 0, 0)),
        out_shape=jax.ShapeDtypeStruct((B, N, 2 * D), jnp.float32),
        interpret=_INTERP,
    )(aggr, tprev, dists, de_pad, uw, ub, w1, w2, b)


_EB = 2048


def _msg_body(g_ref, conf_ref, em_ref, w0_ref, w4_ref, out_ref):
    comp = g_ref[0, :, :D]
    addv = g_ref[0, :, D:]
    x = (jnp.dot(comp, w0_ref[...], preferred_element_type=jnp.float32)
         + jnp.dot(conf_ref[0], w4_ref[...], preferred_element_type=jnp.float32)
         + addv)
    x = jnp.maximum(x, 0.0)
    out_ref[0] = x * em_ref[0, 0][:, None]


def _msg(g, conf, emf, w0, w4):
    return pl.pallas_call(
        _msg_body,
        grid=(B, E // _EB),
        in_specs=[
            pl.BlockSpec((1, _EB, 2 * D), lambda i, j: (i, j, 0)),
            pl.BlockSpec((1, _EB, D), lambda i, j: (i, j, 0)),
            pl.BlockSpec((1, 1, _EB), lambda i, j: (i, 0, j)),
            pl.BlockSpec((D, D), lambda i, j: (0, 0)),
            pl.BlockSpec((D, D), lambda i, j: (0, 0)),
        ],
        out_specs=pl.BlockSpec((1, _EB, D), lambda i, j: (i, j, 0)),
        out_shape=jax.ShapeDtypeStruct((B, E, D), jnp.float32),
        interpret=_INTERP,
    )(g, conf, emf, w0, w4)


def _final_body(aggr_ref, tprev_ref, maskf_ref, rq_ref, attw_ref, attb_ref,
                uw_ref, ub_ref, hout_ref, tout_ref):
    h_prev = tprev_ref[0, :, :D]
    h = (jnp.dot(aggr_ref[0], uw_ref[...], preferred_element_type=jnp.float32)
         + ub_ref[...] + h_prev)
    mf = maskf_ref[0, 0]
    hm = h * mf[:, None]
    att_in = jnp.concatenate(
        [hm, jnp.broadcast_to(rq_ref[0], (N, D))], axis=1)
    s = (jnp.dot(att_in, attw_ref[...],
                 preferred_element_type=jnp.float32)[:, 0]
         + attb_ref[0, 0])
    s = jnp.where(s >= 0, s, 0.01 * s)
    s = jnp.where(mf > 0, s, -1e9)
    smax = jnp.max(s)
    ex = jnp.exp(s - smax)
    alpha = ex / jnp.sum(ex)

    iota = jax.lax.broadcasted_iota(jnp.int32, (N,), 0)
    a = alpha
    rows = []
    for _ in range(M):
        v = jnp.max(a)
        idx = jnp.min(jnp.where(a == v, iota, N))
        rows.append(jnp.where(iota == idx, v, 0.0))
        a = jnp.where(iota == idx, -1.0, a)
    oh = jnp.stack(rows, axis=0)
    hout_ref[0] = jnp.dot(oh, hm, preferred_element_type=jnp.float32)
    tout_ref[0] = hm[0:1, :]


def _final(aggr, tprev, maskf, rq, attw, attb, uw, ub):
    return pl.pallas_call(
        _final_body,
        grid=(B,),
        in_specs=[
            pl.BlockSpec((1, N, D), lambda i: (i, 0, 0)),
            pl.BlockSpec((1, N, 2 * D), lambda i: (i, 0, 0)),
            pl.BlockSpec((1, 1, N), lambda i: (i, 0, 0)),
            pl.BlockSpec((1, 1, D), lambda i: (i, 0, 0)),
            pl.BlockSpec((2 * D, 1), lambda i: (0, 0)),
            pl.BlockSpec((1, 1), lambda i: (0, 0)),
            pl.BlockSpec((D, D), lambda i: (0, 0)),
            pl.BlockSpec((1, D), lambda i: (0, 0)),
        ],
        out_specs=[
            pl.BlockSpec((1, M, D), lambda i: (i, 0, 0)),
            pl.BlockSpec((1, 1, D), lambda i: (i, 0, 0)),
        ],
        out_shape=[
            jax.ShapeDtypeStruct((B, M, D), jnp.float32),
            jax.ShapeDtypeStruct((B, 1, D), jnp.float32),
        ],
        interpret=_INTERP,
    )(aggr, tprev, maskf, rq, attw, attb, uw, ub)


_GC = 128


def _gather_sc(t2, rr2k, srcg, rels):
    mesh = plsc.VectorSubcoreMesh(core_axis_name="c", subcore_axis_name="s", num_cores=2, num_subcores=16)

    @functools.partial(
        pl.kernel, mesh=mesh,
        out_type=jax.ShapeDtypeStruct((B, E, 2 * D), jnp.float32),
        scratch_types=[
            pltpu.VMEM((_GC,), jnp.int32),
            pltpu.VMEM((_GC,), jnp.int32),
            pltpu.VMEM((_GC, 2 * D), jnp.float32),
            pltpu.VMEM((_GC, 2 * D), jnp.float32),
            pltpu.VMEM((_GC, 2 * D), jnp.float32),
            pltpu.SemaphoreType.DMA,
            pltpu.SemaphoreType.DMA,
        ])
    def gk(t2_hbm, rr2_hbm, src_hbm, rels_hbm, out_hbm,
           sidx, ridx, trow, rrow, orow, sem1, sem2):
        wid = jax.lax.axis_index("s") * 2 + jax.lax.axis_index("c")

        def chunk_body(ci, _):
            base = ci * _GC
            pltpu.sync_copy(src_hbm.at[wid, pl.ds(base, _GC)], sidx)
            pltpu.sync_copy(rels_hbm.at[wid, pl.ds(base, _GC)], ridx)
            cp1 = pltpu.async_copy(t2_hbm.at[sidx], trow, sem1)
            cp2 = pltpu.async_copy(rr2_hbm.at[ridx], rrow, sem2)
            cp1.wait()
            cp2.wait()

            def ebody(e, _):
                for j in range(16):
                    a = trow[e, pl.ds(j * 16, 16)]
                    bv = rrow[e, pl.ds(j * 16, 16)]
                    orow[e, pl.ds(j * 16, 16)] = a * bv if j < 8 else a + bv
                return 0

            jax.lax.fori_loop(0, _GC, ebody, 0)
            pltpu.sync_copy(orow, out_hbm.at[wid, pl.ds(base, _GC), :])
            return 0

        jax.lax.fori_loop(0, E // _GC, chunk_body, 0)

    return gk(t2, rr2k, srcg, rels)


def _gather_edges(t, rr2k, src_g, rels):
    t2 = t.reshape(B * N, 2 * D)
    return _gather_sc(t2, rr2k, src_g, rels)


_SCC = 128


def _scatter_sc(msg, tgt):
    mesh = plsc.VectorSubcoreMesh(core_axis_name="c", subcore_axis_name="s", num_cores=2, num_subcores=16)
    epw = E // 16

    @functools.partial(
        pl.kernel, mesh=mesh,
        out_type=jax.ShapeDtypeStruct((B, N, D), jnp.float32),
        scratch_types=[
            pltpu.VMEM((_SCC,), jnp.int32),
            pltpu.VMEM((_SCC, D), jnp.float32),
            pltpu.VMEM((N // 16, D), jnp.float32),
            pltpu.VMEM_SHARED((N, D), jnp.float32),
        ])
    def sk(msg_hbm, tgt_hbm, out_hbm, tidx, mrow, zbuf, acc_sh):
        c = jax.lax.axis_index("c")
        s = jax.lax.axis_index("s")

        def zfill(i, _):
            for j in range(D // 16):
                zbuf[i, pl.ds(j * 16, 16)] = jnp.zeros((16,), jnp.float32)
            return 0

        jax.lax.fori_loop(0, N // 16, zfill, 0)

        def batch_body(i, _):
            b = i * 2 + c
            pltpu.sync_copy(zbuf, acc_sh.at[pl.ds(s * (N // 16), N // 16), :])
            plsc.subcore_barrier()

            def chunk_body(k, _):
                base = s * epw + k * _SCC
                pltpu.sync_copy(tgt_hbm.at[b, pl.ds(base, _SCC)], tidx)
                pltpu.sync_copy(msg_hbm.at[b, pl.ds(base, _SCC), :], mrow)
                pltpu.sync_copy(mrow, acc_sh.at[tidx], add=True)
                return 0

            jax.lax.fori_loop(0, epw // _SCC, chunk_body, 0)
            plsc.subcore_barrier()
            pltpu.sync_copy(acc_sh.at[pl.ds(s * (N // 16), N // 16), :],
                            out_hbm.at[b, pl.ds(s * (N // 16), N // 16), :])
            plsc.subcore_barrier()
            return 0

        jax.lax.fori_loop(0, B // 2, batch_body, 0)

    return sk(msg, tgt)


def _scatter_edges(msg, tgt):
    return _scatter_sc(msg, tgt)


def kernel(dists, edge_index, rels, mask, edge_mask, r_query_embed,
           conf_embeds, params):
    noise = jax.random.normal(jax.random.key(42), (B, N, D),
                              dtype=jnp.float32) * 0.1
    de_pad = jnp.zeros((16, D), jnp.float32).at[:10].set(params['dist_embed'])
    rel_pad = jnp.zeros((512, D), jnp.float32).at[:NREL].set(params['rel_table'])
    dists_c = jnp.clip(dists, 0, 9).astype(jnp.int32)[:, None, :]
    src = edge_index[:, 0, :].astype(jnp.int32)
    tgt = edge_index[:, 1, :].astype(jnp.int32)
    rels_i = rels.astype(jnp.int32)
    emf = edge_mask.astype(jnp.float32)[:, None, :]
    maskf = mask.astype(jnp.float32)[:, None, :]

    w = [params['msg_W'][k] for k in range(NL)]
    w0 = [w[k][:D] for k in range(NL)]
    w1 = [w[k][D:2 * D] for k in range(NL)]
    w2 = [w[k][2 * D:3 * D] for k in range(NL)]
    w3 = jnp.stack([w[k][3 * D:4 * D] for k in range(NL)])
    w4 = [w[k][4 * D:] for k in range(NL)]
    bmsg = [params['msg_b'][k][None, :] for k in range(NL)]
    uw = [params['upd_W'][k] for k in range(NL)]
    ub = [params['upd_b'][k][None, :] for k in range(NL)]

    rr2 = _relprep(rel_pad, w3)

    t = _pre0(dists_c, noise, de_pad, w1[0], w2[0], bmsg[0])
    src_g = src + (jnp.arange(B, dtype=jnp.int32) * N)[:, None]
    for k in range(NL):
        g = _gather_edges(t, rr2[k], src_g, rels_i)
        msg = _msg(g, conf_embeds, emf, w0[k], w4[k])
        aggr = _scatter_edges(msg, tgt)
        if k < NL - 1:
            t = _updpre(aggr, t, dists_c, de_pad, uw[k], ub[k],
                        w1[k + 1], w2[k + 1], bmsg[k + 1])
    h_evd, t_state = _final(aggr, t, maskf, r_query_embed[:, None, :],
                            params['att_W'], params['att_b'][None, :],
                            uw[NL - 1], ub[NL - 1])
    return (h_evd, t_state[:, 0, :])
